# Initial kernel scaffold; baseline (speedup 1.0000x reference)
import jax
import jax.numpy as jnp
from jax.experimental import pallas as pl


def kernel(x_seq, edge_index, edge_attr, W_enc, b_enc, lin0_W, lin0_b, mlp0_W1, mlp0_b1, mlp0_W2, mlp0_b2, ln0_g, ln0_b, lin1_W, lin1_b, mlp1_W1, mlp1_b1, mlp1_W2, mlp1_b2, ln1_g, ln1_b, W_ih, W_hh, b_ih, b_hh, W_head, b_head):
    raise NotImplementedError("write your pallas kernel here")


# speedup vs baseline: 2.5735x; 2.5735x over previous
import functools

import jax
import jax.numpy as jnp
from jax import lax
from jax.experimental import pallas as pl
from jax.experimental.pallas import tpu as pltpu
from jax.experimental.pallas import tpu_sc as plsc

F32 = jnp.float32


def _linear(x, w_t, b, block_rows):
    M, K = x.shape
    Hout = w_t.shape[1]
    nb = M // block_rows

    def body(x_ref, w_ref, b_ref, o_ref):
        o_ref[...] = (
            jnp.dot(x_ref[...], w_ref[...], preferred_element_type=F32) + b_ref[...]
        )

    return pl.pallas_call(
        body,
        grid=(nb,),
        in_specs=[
            pl.BlockSpec((block_rows, K), lambda i: (i, 0)),
            pl.BlockSpec((K, Hout), lambda i: (0, 0)),
            pl.BlockSpec((1, Hout), lambda i: (0, 0)),
        ],
        out_specs=pl.BlockSpec((block_rows, Hout), lambda i: (i, 0)),
        out_shape=jax.ShapeDtypeStruct((M, Hout), F32),
    )(x, w_t, b)


def _edge_embed(attr, w0_t, b0, w1_t, b1, block_rows=2000):
    E, D = attr.shape
    H = w0_t.shape[1]
    nb = E // block_rows

    def body(a_ref, w0_ref, b0_ref, w1_ref, b1_ref, e0_ref, e1_ref):
        a = a_ref[...]
        e0_ref[...] = jnp.dot(a, w0_ref[...], preferred_element_type=F32) + b0_ref[...]
        e1_ref[...] = jnp.dot(a, w1_ref[...], preferred_element_type=F32) + b1_ref[...]

    return pl.pallas_call(
        body,
        grid=(nb,),
        in_specs=[
            pl.BlockSpec((block_rows, D), lambda i: (i, 0)),
            pl.BlockSpec((D, H), lambda i: (0, 0)),
            pl.BlockSpec((1, H), lambda i: (0, 0)),
            pl.BlockSpec((D, H), lambda i: (0, 0)),
            pl.BlockSpec((1, H), lambda i: (0, 0)),
        ],
        out_specs=[
            pl.BlockSpec((block_rows, H), lambda i: (i, 0)),
            pl.BlockSpec((block_rows, H), lambda i: (i, 0)),
        ],
        out_shape=[
            jax.ShapeDtypeStruct((E, H), F32),
            jax.ShapeDtypeStruct((E, H), F32),
        ],
    )(attr, w0_t, b0, w1_t, b1)


def _post(x, agg, w1_t, b1, w2_t, b2, g, b, block_rows=2000):
    M, H = x.shape
    nb = M // block_rows

    def body(x_ref, a_ref, w1_ref, b1_ref, w2_ref, b2_ref, g_ref, bb_ref, o_ref):
        x_ = x_ref[...]
        h = x_ + a_ref[...]
        y = jnp.maximum(
            jnp.dot(h, w1_ref[...], preferred_element_type=F32) + b1_ref[...], 0.0
        )
        y = jnp.dot(y, w2_ref[...], preferred_element_type=F32) + b2_ref[...]
        mu = jnp.mean(y, axis=-1, keepdims=True)
        var = jnp.mean((y - mu) ** 2, axis=-1, keepdims=True)
        z = (y - mu) * lax.rsqrt(var + 1e-5) * g_ref[...] + bb_ref[...]
        o_ref[...] = jnp.maximum(z, 0.0) + x_

    full = lambda i: (0, 0)
    return pl.pallas_call(
        body,
        grid=(nb,),
        in_specs=[
            pl.BlockSpec((block_rows, H), lambda i: (i, 0)),
            pl.BlockSpec((block_rows, H), lambda i: (i, 0)),
            pl.BlockSpec((H, H), full),
            pl.BlockSpec((1, H), full),
            pl.BlockSpec((H, H), full),
            pl.BlockSpec((1, H), full),
            pl.BlockSpec((1, H), full),
            pl.BlockSpec((1, H), full),
        ],
        out_specs=pl.BlockSpec((block_rows, H), lambda i: (i, 0)),
        out_shape=jax.ShapeDtypeStruct((M, H), F32),
    )(x, agg, w1_t, b1, w2_t, b2, g, b)


def _gru_head(seq, wih_t, whh_t, bih, bhh, w_head, b_head, block_rows=1024):
    T, Np, H = seq.shape
    nb = Np // block_rows

    def body(s_ref, wih_ref, whh_ref, bih_ref, bhh_ref, wh_ref, bh_ref, o_ref):
        h = jnp.zeros((block_rows, H), F32)
        for t in range(T):
            xt = s_ref[t]
            gx = jnp.dot(xt, wih_ref[...], preferred_element_type=F32) + bih_ref[...]
            gh = jnp.dot(h, whh_ref[...], preferred_element_type=F32) + bhh_ref[...]
            r = jax.nn.sigmoid(gx[:, :H] + gh[:, :H])
            z = jax.nn.sigmoid(gx[:, H : 2 * H] + gh[:, H : 2 * H])
            n = jnp.tanh(gx[:, 2 * H :] + r * gh[:, 2 * H :])
            h = (1.0 - z) * n + z * h
        p = jnp.sum(h * wh_ref[...], axis=1, keepdims=True) + bh_ref[0, 0]
        o_ref[...] = jnp.broadcast_to(p, (block_rows, H))

    full = lambda i: (0, 0)
    return pl.pallas_call(
        body,
        grid=(nb,),
        in_specs=[
            pl.BlockSpec((T, block_rows, H), lambda i: (0, i, 0)),
            pl.BlockSpec((H, 3 * H), full),
            pl.BlockSpec((H, 3 * H), full),
            pl.BlockSpec((1, 3 * H), full),
            pl.BlockSpec((1, 3 * H), full),
            pl.BlockSpec((1, H), full),
            pl.BlockSpec((1, 1), full),
        ],
        out_specs=pl.BlockSpec((block_rows, H), lambda i: (i, 0)),
        out_shape=jax.ShapeDtypeStruct((Np, H), F32),
    )(seq, wih_t, whh_t, bih, bhh, w_head, b_head)


def _message(x_flat, src, dst, e, zeros_blk, T, N, H):
    E = src.shape[0]
    NSUB = 16
    NCORE = 2
    TP = T // NCORE
    EPT = E // NSUB
    CH = 80
    NCH = EPT // CH
    NP = ((N + 127) // 128) * 128
    NPS = NP // NSUB

    mesh = plsc.VectorSubcoreMesh(
        core_axis_name="c", subcore_axis_name="s", num_cores=NCORE, num_subcores=NSUB
    )

    @functools.partial(
        pl.kernel,
        out_type=jax.ShapeDtypeStruct((T * NP, H), F32),
        mesh=mesh,
        scratch_types=[
            pltpu.VMEM((CH,), jnp.int32),
            pltpu.VMEM((CH,), jnp.int32),
            pltpu.VMEM((CH, H), F32),
            pltpu.VMEM((CH, H), F32),
            pltpu.VMEM_SHARED((NP, H), F32),
            pltpu.SemaphoreType.DMA,
        ],
    )
    def msg(x_hbm, src_hbm, dst_hbm, e_hbm, z_hbm, out_hbm, src_v, dst_v, e_v, xr_v, acc, sem):
        c = lax.axis_index("c")
        s = lax.axis_index("s")
        ebase = s * EPT
        for j in range(TP):
            t = c * TP + j
            pltpu.sync_copy(z_hbm, acc.at[pl.ds(s * NPS, NPS)])
            plsc.subcore_barrier()

            def chunk(ci, _):
                off = ebase + ci * CH
                pltpu.sync_copy(src_hbm.at[pl.ds(off, CH)], src_v)
                pltpu.sync_copy(dst_hbm.at[pl.ds(off, CH)], dst_v)
                pltpu.sync_copy(e_hbm.at[pl.ds(off, CH)], e_v)
                for g in range(CH // 16):
                    sl = pl.ds(g * 16, 16)
                    src_v[sl] = src_v[sl] + t * N
                pltpu.async_copy(x_hbm.at[src_v], xr_v, sem).wait()

                def row(i, _):
                    for k in range(H // 16):
                        cs = pl.ds(k * 16, 16)
                        xr_v[i, cs] = jnp.maximum(xr_v[i, cs] + e_v[i, cs], 0.0)
                    return 0

                lax.fori_loop(0, CH, row, 0)
                pltpu.sync_copy(xr_v, acc.at[dst_v], add=True)
                return 0

            lax.fori_loop(0, NCH, chunk, 0)
            plsc.subcore_barrier()
            pltpu.sync_copy(
                acc.at[pl.ds(s * NPS, NPS)],
                out_hbm.at[pl.ds(t * NP + s * NPS, NPS)],
            )

    out = msg(x_flat, src, dst, e, zeros_blk)
    return out.reshape(T, NP, H)[:, :N].reshape(T * N, H)


def kernel(x_seq, edge_index, edge_attr, W_enc, b_enc, lin0_W, lin0_b, mlp0_W1,
           mlp0_b1, mlp0_W2, mlp0_b2, ln0_g, ln0_b, lin1_W, lin1_b, mlp1_W1,
           mlp1_b1, mlp1_W2, mlp1_b2, ln1_g, ln1_b, W_ih, W_hh, b_ih, b_hh,
           W_head, b_head):
    B, T, N, F = x_seq.shape
    H = W_enc.shape[0]
    E = edge_index.shape[1]
    src = edge_index[0]
    dst = edge_index[1]

    r2 = lambda v: v.reshape(1, -1)

    e0, e1 = _edge_embed(edge_attr, lin0_W.T, r2(lin0_b), lin1_W.T, r2(lin1_b))
    X = _linear(x_seq.reshape(T * N, F), W_enc.T, r2(b_enc), block_rows=2000)

    zeros_blk = jnp.zeros((((N + 127) // 128) * 128 // 16, H), F32)

    agg0 = _message(X, src, dst, e0, zeros_blk, T, N, H)
    X1 = _post(X, agg0, mlp0_W1.T, r2(mlp0_b1), mlp0_W2.T, r2(mlp0_b2),
               r2(ln0_g), r2(ln0_b))
    agg1 = _message(X1, src, dst, e1, zeros_blk, T, N, H)
    X2 = _post(X1, agg1, mlp1_W1.T, r2(mlp1_b1), mlp1_W2.T, r2(mlp1_b2),
               r2(ln1_g), r2(ln1_b))

    Np = ((N + 1023) // 1024) * 1024
    seq = jnp.pad(X2.reshape(T, N, H), ((0, 0), (0, Np - N), (0, 0)))
    hout = _gru_head(seq, W_ih.T, W_hh.T, r2(b_ih), r2(b_hh), W_head,
                     b_head.reshape(1, 1))
    return hout[:N, 0].reshape(1, N)


# speedup vs baseline: 4.7260x; 1.8364x over previous
import functools

import jax
import jax.numpy as jnp
from jax import lax
from jax.experimental import pallas as pl
from jax.experimental.pallas import tpu as pltpu
from jax.experimental.pallas import tpu_sc as plsc

F32 = jnp.float32


def _linear(x, w_t, b, block_rows):
    M, K = x.shape
    Hout = w_t.shape[1]
    nb = M // block_rows

    def body(x_ref, w_ref, b_ref, o_ref):
        o_ref[...] = (
            jnp.dot(x_ref[...], w_ref[...], preferred_element_type=F32) + b_ref[...]
        )

    return pl.pallas_call(
        body,
        grid=(nb,),
        in_specs=[
            pl.BlockSpec((block_rows, K), lambda i: (i, 0)),
            pl.BlockSpec((K, Hout), lambda i: (0, 0)),
            pl.BlockSpec((1, Hout), lambda i: (0, 0)),
        ],
        out_specs=pl.BlockSpec((block_rows, Hout), lambda i: (i, 0)),
        out_shape=jax.ShapeDtypeStruct((M, Hout), F32),
    )(x, w_t, b)


def _edge_embed(attr, w0_t, b0, w1_t, b1, block_rows=2000):
    E, D = attr.shape
    H = w0_t.shape[1]
    nb = E // block_rows

    def body(a_ref, w0_ref, b0_ref, w1_ref, b1_ref, e0_ref, e1_ref):
        a = a_ref[...]
        e0_ref[...] = jnp.dot(a, w0_ref[...], preferred_element_type=F32) + b0_ref[...]
        e1_ref[...] = jnp.dot(a, w1_ref[...], preferred_element_type=F32) + b1_ref[...]

    return pl.pallas_call(
        body,
        grid=(nb,),
        in_specs=[
            pl.BlockSpec((block_rows, D), lambda i: (i, 0)),
            pl.BlockSpec((D, H), lambda i: (0, 0)),
            pl.BlockSpec((1, H), lambda i: (0, 0)),
            pl.BlockSpec((D, H), lambda i: (0, 0)),
            pl.BlockSpec((1, H), lambda i: (0, 0)),
        ],
        out_specs=[
            pl.BlockSpec((block_rows, H), lambda i: (i, 0)),
            pl.BlockSpec((block_rows, H), lambda i: (i, 0)),
        ],
        out_shape=[
            jax.ShapeDtypeStruct((E, H), F32),
            jax.ShapeDtypeStruct((E, H), F32),
        ],
    )(attr, w0_t, b0, w1_t, b1)


def _post(x, agg, w1_t, b1, w2_t, b2, g, b, block_rows=2000):
    M, H = x.shape
    nb = M // block_rows

    def body(x_ref, a_ref, w1_ref, b1_ref, w2_ref, b2_ref, g_ref, bb_ref, o_ref):
        x_ = x_ref[...]
        h = x_ + a_ref[...]
        y = jnp.maximum(
            jnp.dot(h, w1_ref[...], preferred_element_type=F32) + b1_ref[...], 0.0
        )
        y = jnp.dot(y, w2_ref[...], preferred_element_type=F32) + b2_ref[...]
        mu = jnp.mean(y, axis=-1, keepdims=True)
        var = jnp.mean((y - mu) ** 2, axis=-1, keepdims=True)
        z = (y - mu) * lax.rsqrt(var + 1e-5) * g_ref[...] + bb_ref[...]
        o_ref[...] = jnp.maximum(z, 0.0) + x_

    full = lambda i: (0, 0)
    return pl.pallas_call(
        body,
        grid=(nb,),
        in_specs=[
            pl.BlockSpec((block_rows, H), lambda i: (i, 0)),
            pl.BlockSpec((block_rows, H), lambda i: (i, 0)),
            pl.BlockSpec((H, H), full),
            pl.BlockSpec((1, H), full),
            pl.BlockSpec((H, H), full),
            pl.BlockSpec((1, H), full),
            pl.BlockSpec((1, H), full),
            pl.BlockSpec((1, H), full),
        ],
        out_specs=pl.BlockSpec((block_rows, H), lambda i: (i, 0)),
        out_shape=jax.ShapeDtypeStruct((M, H), F32),
    )(x, agg, w1_t, b1, w2_t, b2, g, b)


def _gru_head(seq, wih_t, whh_t, bih, bhh, w_head, b_head, block_rows=1024):
    T, Np, H = seq.shape
    nb = Np // block_rows

    def body(s_ref, wih_ref, whh_ref, bih_ref, bhh_ref, wh_ref, bh_ref, o_ref):
        h = jnp.zeros((block_rows, H), F32)
        for t in range(T):
            xt = s_ref[t]
            gx = jnp.dot(xt, wih_ref[...], preferred_element_type=F32) + bih_ref[...]
            gh = jnp.dot(h, whh_ref[...], preferred_element_type=F32) + bhh_ref[...]
            r = jax.nn.sigmoid(gx[:, :H] + gh[:, :H])
            z = jax.nn.sigmoid(gx[:, H : 2 * H] + gh[:, H : 2 * H])
            n = jnp.tanh(gx[:, 2 * H :] + r * gh[:, 2 * H :])
            h = (1.0 - z) * n + z * h
        p = jnp.sum(h * wh_ref[...], axis=1, keepdims=True) + bh_ref[0, 0]
        o_ref[...] = jnp.broadcast_to(p, (block_rows, H))

    full = lambda i: (0, 0)
    return pl.pallas_call(
        body,
        grid=(nb,),
        in_specs=[
            pl.BlockSpec((T, block_rows, H), lambda i: (0, i, 0)),
            pl.BlockSpec((H, 3 * H), full),
            pl.BlockSpec((H, 3 * H), full),
            pl.BlockSpec((1, 3 * H), full),
            pl.BlockSpec((1, 3 * H), full),
            pl.BlockSpec((1, H), full),
            pl.BlockSpec((1, 1), full),
        ],
        out_specs=pl.BlockSpec((block_rows, H), lambda i: (i, 0)),
        out_shape=jax.ShapeDtypeStruct((Np, H), F32),
    )(seq, wih_t, whh_t, bih, bhh, w_head, b_head)


def _message(x_flat, src, dst, e, zeros_blk, T, N, H):
    E = src.shape[0]
    NSUB = 16
    NCORE = 2
    TP = T // NCORE
    CH = 64
    GCH = E // CH
    NCH = (GCH + NSUB - 1) // NSUB
    NCH = NCH + (NCH & 1)
    NP = ((N + 127) // 128) * 128
    NPS = NP // NSUB
    DUMP = NP - 8

    mesh = plsc.VectorSubcoreMesh(
        core_axis_name="c", subcore_axis_name="s", num_cores=NCORE, num_subcores=NSUB
    )

    @functools.partial(
        pl.kernel,
        out_type=jax.ShapeDtypeStruct((T * NP, H), F32),
        mesh=mesh,
        scratch_types=[
            [pltpu.VMEM((CH,), jnp.int32)] * 2,
            [pltpu.VMEM((CH,), jnp.int32)] * 2,
            [pltpu.VMEM((CH,), jnp.int32)] * 2,
            [pltpu.VMEM((CH, H), F32)] * 2,
            [pltpu.VMEM((CH, H), F32)] * 2,
            pltpu.VMEM_SHARED((NP, H), F32),
            [pltpu.SemaphoreType.DMA] * 2,
            [pltpu.SemaphoreType.DMA] * 2,
            [pltpu.SemaphoreType.DMA] * 2,
        ],
    )
    def msg(x_hbm, src_hbm, dst_hbm, e_hbm, z_hbm, out_hbm, src_v, dst_v, dsc_v,
            e_v, xr_v, acc, fsem, gsem, ssem):
        c = lax.axis_index("c")
        s = lax.axis_index("s")

        def chunk_off(k):
            g = k * NSUB + s
            g = jnp.minimum(g, GCH - 1)
            return g * CH

        def fetch(k, b):
            off = chunk_off(k)
            pltpu.async_copy(src_hbm.at[pl.ds(off, CH)], src_v[b], fsem[b])
            pltpu.async_copy(dst_hbm.at[pl.ds(off, CH)], dst_v[b], fsem[b])
            pltpu.async_copy(e_hbm.at[pl.ds(off, CH)], e_v[b], fsem[b])

        def wait_fetch(b):
            pltpu.make_async_copy(src_hbm.at[pl.ds(0, CH)], src_v[b], fsem[b]).wait()
            pltpu.make_async_copy(dst_hbm.at[pl.ds(0, CH)], dst_v[b], fsem[b]).wait()
            pltpu.make_async_copy(e_hbm.at[pl.ds(0, CH)], e_v[b], fsem[b]).wait()

        def prep_idx(k, b, t):
            fake = (k * NSUB + s) >= GCH
            for g in range(CH // 16):
                sl = pl.ds(g * 16, 16)
                src_v[b][sl] = src_v[b][sl] + t * N
                dst_v[b][sl] = jnp.where(fake, DUMP, dst_v[b][sl])

        def gather(b):
            pltpu.async_copy(x_hbm.at[src_v[b]], xr_v[b], gsem[b])

        def wait_gather(b):
            pltpu.make_async_copy(x_hbm.at[src_v[b]], xr_v[b], gsem[b]).wait()

        def wait_scatter(b):
            pltpu.make_async_copy(xr_v[b], acc.at[dsc_v[b]], ssem[b]).wait()

        for j in range(TP):
            t = c * TP + j
            pltpu.sync_copy(z_hbm, acc.at[pl.ds(s * NPS, NPS)])
            plsc.subcore_barrier()

            fetch(0, 0)
            fetch(1, 1)
            wait_fetch(0)
            prep_idx(0, 0, t)
            gather(0)

            def step(ci2, _):
                for b in (0, 1):
                    ci = ci2 * 2 + b
                    nb = 1 - b

                    @pl.when(ci + 1 < NCH)
                    def _():
                        wait_fetch(nb)
                        prep_idx(ci + 1, nb, t)

                    @pl.when(ci >= 1)
                    def _():
                        wait_scatter(nb)

                    @pl.when(ci + 1 < NCH)
                    def _():
                        gather(nb)

                    wait_gather(b)

                    def row(i, _):
                        for kk in range(H // 16):
                            cs = pl.ds(kk * 16, 16)
                            xr_v[b][i, cs] = jnp.maximum(
                                xr_v[b][i, cs] + e_v[b][i, cs], 0.0
                            )
                        return 0

                    lax.fori_loop(0, CH, row, 0)
                    for g in range(CH // 16):
                        sl = pl.ds(g * 16, 16)
                        dsc_v[b][sl] = dst_v[b][sl]
                    pltpu.async_copy(xr_v[b], acc.at[dsc_v[b]], ssem[b], add=True)

                    @pl.when(ci + 2 < NCH)
                    def _():
                        fetch(ci + 2, b)

                return 0

            lax.fori_loop(0, NCH // 2, step, 0)
            wait_scatter(1)
            plsc.subcore_barrier()
            pltpu.sync_copy(
                acc.at[pl.ds(s * NPS, NPS)],
                out_hbm.at[pl.ds(t * NP + s * NPS, NPS)],
            )

    out = msg(x_flat, src, dst, e, zeros_blk)
    return out.reshape(T, NP, H)[:, :N].reshape(T * N, H)


def kernel(x_seq, edge_index, edge_attr, W_enc, b_enc, lin0_W, lin0_b, mlp0_W1,
           mlp0_b1, mlp0_W2, mlp0_b2, ln0_g, ln0_b, lin1_W, lin1_b, mlp1_W1,
           mlp1_b1, mlp1_W2, mlp1_b2, ln1_g, ln1_b, W_ih, W_hh, b_ih, b_hh,
           W_head, b_head):
    B, T, N, F = x_seq.shape
    H = W_enc.shape[0]
    E = edge_index.shape[1]
    src = edge_index[0]
    dst = edge_index[1]

    r2 = lambda v: v.reshape(1, -1)

    e0, e1 = _edge_embed(edge_attr, lin0_W.T, r2(lin0_b), lin1_W.T, r2(lin1_b))
    X = _linear(x_seq.reshape(T * N, F), W_enc.T, r2(b_enc), block_rows=2000)

    zeros_blk = jnp.zeros((((N + 127) // 128) * 128 // 16, H), F32)

    agg0 = _message(X, src, dst, e0, zeros_blk, T, N, H)
    X1 = _post(X, agg0, mlp0_W1.T, r2(mlp0_b1), mlp0_W2.T, r2(mlp0_b2),
               r2(ln0_g), r2(ln0_b))
    agg1 = _message(X1, src, dst, e1, zeros_blk, T, N, H)
    X2 = _post(X1, agg1, mlp1_W1.T, r2(mlp1_b1), mlp1_W2.T, r2(mlp1_b2),
               r2(ln1_g), r2(ln1_b))

    Np = ((N + 1023) // 1024) * 1024
    seq = jnp.pad(X2.reshape(T, N, H), ((0, 0), (0, Np - N), (0, 0)))
    hout = _gru_head(seq, W_ih.T, W_hh.T, r2(b_ih), r2(b_hh), W_head,
                     b_head.reshape(1, 1))
    return hout[:N, 0].reshape(1, N)
